# Initial kernel scaffold; baseline (speedup 1.0000x reference)
import jax
import jax.numpy as jnp
from jax.experimental import pallas as pl


def kernel(ind, feature, feature_bank, label_bank, centroids):
    raise NotImplementedError("write your pallas kernel here")


# speedup vs baseline: 1.6098x; 1.6098x over previous
import functools

import jax
import jax.numpy as jnp
from jax import lax
from jax.experimental import pallas as pl
from jax.experimental.pallas import tpu as pltpu
from jax.experimental.pallas import tpu_sc as plsc

_LENGTH = 1000000
_FEAT = 64
_NCLS = 1000
_B = 16384
_MOM = 0.5

_NC = 2
_NS = 16
_NW = _NC * _NS
_CH = _B // _NW
_PIECE = 128
_NP = _CH // _PIECE

def _worker_base():
  wid = lax.axis_index("s") * _NC + lax.axis_index("c")
  return wid * _CH


def _sc_gather_body(ind_hbm, fbank_hbm, lbank_hbm, fold_hbm, lold_hbm,
                    idx2, rows_v, lab2, sem):
  base = _worker_base()
  for j in range(_NP):
    pltpu.sync_copy(ind_hbm.at[pl.ds(base + j * _PIECE, _PIECE)], idx2.at[j])
  for j in range(_NP):
    pltpu.async_copy(fbank_hbm.at[idx2.at[j]],
                     rows_v.at[pl.ds(j * _PIECE, _PIECE), :], sem).wait()
    pltpu.async_copy(lbank_hbm.at[idx2.at[j]], lab2.at[j], sem).wait()
  pltpu.sync_copy(rows_v, fold_hbm.at[pl.ds(base, _CH)])
  for j in range(_NP):
    pltpu.sync_copy(lab2.at[j], lold_hbm.at[pl.ds(base + j * _PIECE, _PIECE)])


@functools.cache
def _get_sc_gather():
  return pl.kernel(
      _sc_gather_body,
      out_type=(jax.ShapeDtypeStruct((_B, _FEAT), jnp.float32),
                jax.ShapeDtypeStruct((_B,), jnp.int32)),
      mesh=plsc.VectorSubcoreMesh(core_axis_name="c", subcore_axis_name="s"),
      compiler_params=pltpu.CompilerParams(use_tc_tiling_on_sc=False),
      scratch_types=[
          pltpu.VMEM((_NP, _PIECE), jnp.int32),
          pltpu.VMEM((_CH, _FEAT), jnp.float32),
          pltpu.VMEM((_NP, _PIECE), jnp.int32),
          pltpu.SemaphoreType.DMA,
      ],
  )


def _sc_scatter_body(ind_hbm, win_hbm, v2_hbm, nl_hbm, fb_ref, lb_ref,
                     idx2, src2, rows_v, lab2, sem):
  base = _worker_base()
  for j in range(_NP):
    pltpu.sync_copy(ind_hbm.at[pl.ds(base + j * _PIECE, _PIECE)], idx2.at[j])
  for j in range(_NP):
    pltpu.async_copy(win_hbm.at[idx2.at[j]], src2.at[j], sem).wait()
  for j in range(_NP):
    pltpu.async_copy(v2_hbm.at[src2.at[j]],
                     rows_v.at[pl.ds(j * _PIECE, _PIECE), :], sem).wait()
    pltpu.async_copy(nl_hbm.at[src2.at[j]], lab2.at[j], sem).wait()
  for j in range(_NP):
    pltpu.async_copy(rows_v.at[pl.ds(j * _PIECE, _PIECE), :],
                     fb_ref.at[idx2.at[j]], sem).wait()
    pltpu.async_copy(lab2.at[j], lb_ref.at[idx2.at[j]], sem).wait()


@functools.cache
def _get_sc_scatter():
  return pl.kernel(
      _sc_scatter_body,
      out_type=(),
      mesh=plsc.VectorSubcoreMesh(core_axis_name="c", subcore_axis_name="s"),
      compiler_params=pltpu.CompilerParams(use_tc_tiling_on_sc=False),
      scratch_types=[
          pltpu.VMEM((_NP, _PIECE), jnp.int32),
          pltpu.VMEM((_NP, _PIECE), jnp.int32),
          pltpu.VMEM((_CH, _FEAT), jnp.float32),
          pltpu.VMEM((_NP, _PIECE), jnp.int32),
          pltpu.SemaphoreType.DMA,
      ],
  )


_TC_BLK = 1024
_TC_GRID = _B // _TC_BLK


def _tc_body(feat_ref, fold_ref, lold_ref, cents_ref, v2_ref, nl_ref, cnt_ref):
  i = pl.program_id(0)
  f = feat_ref[...]
  fo = fold_ref[...]
  fn = f / (jnp.sqrt(jnp.sum(f * f, axis=1, keepdims=True)) + 1e-10)
  fnew = (1.0 - _MOM) * fo + _MOM * fn
  v2 = fnew / (jnp.sqrt(jnp.sum(fnew * fnew, axis=1, keepdims=True)) + 1e-10)
  v2_ref[...] = v2
  sim = lax.dot_general(v2, cents_ref[...], (((1,), (1,)), ((), ())),
                        preferred_element_type=jnp.float32)
  m = jnp.max(sim, axis=1, keepdims=True)
  cls_iota = lax.broadcasted_iota(jnp.int32, sim.shape, 1)
  lbl = jnp.min(jnp.where(sim >= m, cls_iota, _NCLS), axis=1).astype(jnp.int32)
  nl_ref[...] = lbl
  changed = jnp.sum((lbl != lold_ref[...]).astype(jnp.float32))

  @pl.when(i == 0)
  def _():
    cnt_ref[0, 0] = 0.0

  cnt_ref[0, 0] += changed


_tc_compute = pl.pallas_call(
    _tc_body,
    grid=(_TC_GRID,),
    in_specs=[
        pl.BlockSpec((_TC_BLK, _FEAT), lambda i: (i, 0)),
        pl.BlockSpec((_TC_BLK, _FEAT), lambda i: (i, 0)),
        pl.BlockSpec((_TC_BLK,), lambda i: (i,)),
        pl.BlockSpec((_NCLS, _FEAT), lambda i: (0, 0)),
    ],
    out_specs=[
        pl.BlockSpec((_TC_BLK, _FEAT), lambda i: (i, 0)),
        pl.BlockSpec((_TC_BLK,), lambda i: (i,)),
        pl.BlockSpec(memory_space=pltpu.SMEM, block_shape=(1, 1),
                     index_map=lambda i: (0, 0)),
    ],
    out_shape=[
        jax.ShapeDtypeStruct((_B, _FEAT), jnp.float32),
        jax.ShapeDtypeStruct((_B,), jnp.int32),
        jax.ShapeDtypeStruct((1, 1), jnp.float32),
    ],
)


def kernel(ind, feature, feature_bank, label_bank, centroids):
  ind = ind.astype(jnp.int32)
  fold, lold = _get_sc_gather()(ind, feature_bank, label_bank)
  v2, nl, cnt = _tc_compute(feature, fold, lold, centroids)
  pos = jnp.arange(_B, dtype=jnp.int32)
  win = jnp.zeros((_LENGTH,), jnp.int32).at[ind].max(pos)
  fb_ref = jax.new_ref(feature_bank)
  lb_ref = jax.new_ref(label_bank)
  _get_sc_scatter()(ind, win, v2, nl, fb_ref, lb_ref)
  change_ratio = cnt[0, 0] * (1.0 / _B)
  return change_ratio, fb_ref[...], lb_ref[...]
